# baseline (speedup 1.0000x reference)
import functools

import jax
import jax.numpy as jnp
from jax import lax
from jax.experimental import pallas as pl
from jax.experimental.pallas import tpu as pltpu
from jax.experimental.pallas import tpu_sc as plsc

_D_MODEL = 4096
_N_EXP = 64
_K = 8
_BLOCK = 1024
_SPLIT = 2
_SUB = _BLOCK // _SPLIT

_NC = 2
_NS = 16
_NW = _NC * _NS
_L = 16


def _matmul_kernel(*refs):
    x_refs = refs[:_SPLIT]
    w_ref, b_ref, rw_ref = refs[_SPLIT:]
    w = w_ref[...]
    bias = b_ref[...]
    for s in range(_SPLIT):
        rw = lax.dot_general(
            x_refs[s][...], w, (((1,), (1,)), ((), ())),
            preferred_element_type=jnp.float32,
        ) + bias
        rw_ref[pl.ds(s * _SUB, _SUB), :] = rw


def _tc_routing_weights(x, W, b, row0, n_rows):
    grid = (n_rows // _BLOCK,)
    b2 = b.reshape(1, _N_EXP)
    sub0 = row0 // _SUB

    def _x_spec(s):
        return pl.BlockSpec(
            (_SUB, _D_MODEL), lambda i, s=s: (sub0 + _SPLIT * i + s, 0)
        )

    return pl.pallas_call(
        _matmul_kernel,
        grid=grid,
        in_specs=[_x_spec(s) for s in range(_SPLIT)] + [
            pl.BlockSpec((_N_EXP, _D_MODEL), lambda i: (0, 0)),
            pl.BlockSpec((1, _N_EXP), lambda i: (0, 0)),
        ],
        out_specs=pl.BlockSpec((_BLOCK, _N_EXP), lambda i: (i, 0)),
        out_shape=jax.ShapeDtypeStruct((n_rows, _N_EXP), jnp.float32),
        compiler_params=pltpu.CompilerParams(
            dimension_semantics=("parallel",),
        ),
    )(*([x] * _SPLIT + [W, b2]))


def _bcast_max(v):
    return plsc.cummax(lax.rev(plsc.cummax(v), (0,)))


def _sc_gates(rw_flat, n_tokens):
    rows_per_w = n_tokens // _NW
    chunk = rows_per_w * _N_EXP
    mesh = plsc.VectorSubcoreMesh(core_axis_name="c", subcore_axis_name="s")

    @functools.partial(
        pl.kernel,
        mesh=mesh,
        out_type=jax.ShapeDtypeStruct((n_tokens * _N_EXP,), jnp.float32),
        scratch_types=[
            pltpu.VMEM((chunk,), jnp.float32),
            pltpu.VMEM((chunk,), jnp.float32),
        ],
        compiler_params=pltpu.CompilerParams(needs_layout_passes=False),
    )
    def _router(rw_hbm, gates_hbm, buf_in, buf_out):
        wid = lax.axis_index("s") * _NC + lax.axis_index("c")
        base = wid * chunk
        pltpu.sync_copy(rw_hbm.at[pl.ds(base, chunk)], buf_in)

        lane = lax.iota(jnp.int32, _L)
        mask8 = lane < _K
        zero = jnp.zeros((_L,), jnp.float32)
        neg_inf = jnp.float32(-jnp.inf)

        def _sortd(v):
            return plsc.sort_key_val(v, v, descending=True)[0]

        def _merge(p, q):
            return _sortd(jnp.where(mask8, p, lax.rev(q, (0,))))

        def row_body(r, carry):
            off = r * _N_EXP
            v = [buf_in[pl.ds(off + j * _L, _L)] for j in range(4)]
            s0, s1, s2, s3 = (_sortd(vj) for vj in v)
            m3 = _merge(_merge(s0, s1), _merge(s2, s3))

            v0b = plsc.cummax(m3)
            t = jnp.where(mask8, -m3, neg_inf)
            v8b = -_bcast_max(t)

            e = jnp.where(mask8, jnp.exp(m3 - v0b), zero)
            s_e = plsc.cumsum(e)
            zb = plsc.cummax(lax.rev(s_e, (0,)))

            gt = [vj > v8b for vj in v]
            pc = [plsc.all_reduce_population_count(g) for g in gt]
            n_gt = pc[0] + pc[1] + pc[2] + pc[3]
            c = jnp.full((_L,), _K, jnp.int32) - n_gt

            eq = [vj == v8b for vj in v]
            pe = [plsc.all_reduce_population_count(q) for q in eq]
            prev = jnp.zeros((_L,), jnp.int32)
            for j in range(4):
                cs = plsc.cumsum(eq[j].astype(jnp.int32))
                keep = gt[j] | (eq[j] & ((prev + cs) <= c))
                out_j = jnp.where(keep, jnp.exp(v[j] - v0b) / zb, zero)
                buf_out[pl.ds(off + j * _L, _L)] = out_j
                prev = prev + pe[j]
            return carry

        lax.fori_loop(0, rows_per_w, row_body, 0)
        pltpu.sync_copy(buf_out, gates_hbm.at[pl.ds(base, chunk)])

    return _router(rw_flat)


def kernel(x, W, b):
    n_tokens = x.shape[0]
    n_chunks = 4
    c_rows = n_tokens // n_chunks
    rw_parts = []
    gate_parts = []
    for c in range(n_chunks):
        rw_c = _tc_routing_weights(x, W, b, c * c_rows, c_rows)
        rw_parts.append(rw_c)
        gate_parts.append(_sc_gates(rw_c.reshape(-1), c_rows))
    rw = jnp.concatenate(rw_parts, axis=0)
    gates = jnp.concatenate(gate_parts, axis=0).reshape(n_tokens, _N_EXP)
    return (gates, rw)


# speedup vs baseline: 1.2006x; 1.2006x over previous
import functools

import jax
import jax.numpy as jnp
from jax import lax
from jax.experimental import pallas as pl
from jax.experimental.pallas import tpu as pltpu
from jax.experimental.pallas import tpu_sc as plsc

_D_MODEL = 4096
_N_EXP = 64
_K = 8
_BLOCK = 1024
_SPLIT = 2
_SUB = _BLOCK // _SPLIT

_NC = 2
_NS = 16
_NW = _NC * _NS
_L = 16


def _matmul_kernel(*refs):
    x_refs = refs[:_SPLIT]
    w_ref, b_ref, rw_ref = refs[_SPLIT:]
    w = w_ref[...]
    bias = b_ref[...]
    for s in range(_SPLIT):
        rw = lax.dot_general(
            x_refs[s][...], w, (((1,), (1,)), ((), ())),
            preferred_element_type=jnp.float32,
        ) + bias
        rw_ref[pl.ds(s * _SUB, _SUB), :] = rw


def _tc_routing_weights(x, W, b, row0, n_rows):
    grid = (n_rows // _BLOCK,)
    b2 = b.reshape(1, _N_EXP)
    sub0 = row0 // _SUB

    def _x_spec(s):
        return pl.BlockSpec(
            (_SUB, _D_MODEL), lambda i, s=s: (sub0 + _SPLIT * i + s, 0)
        )

    return pl.pallas_call(
        _matmul_kernel,
        grid=grid,
        in_specs=[_x_spec(s) for s in range(_SPLIT)] + [
            pl.BlockSpec((_N_EXP, _D_MODEL), lambda i: (0, 0)),
            pl.BlockSpec((1, _N_EXP), lambda i: (0, 0)),
        ],
        out_specs=pl.BlockSpec((_BLOCK, _N_EXP), lambda i: (i, 0)),
        out_shape=jax.ShapeDtypeStruct((n_rows, _N_EXP), jnp.float32),
        compiler_params=pltpu.CompilerParams(
            dimension_semantics=("parallel",),
        ),
    )(*([x] * _SPLIT + [W, b2]))


def _bcast_max(v):
    return plsc.cummax(lax.rev(plsc.cummax(v), (0,)))


def _sc_gates(rw_flat, n_tokens):
    rows_per_w = n_tokens // _NW
    chunk = rows_per_w * _N_EXP
    mesh = plsc.VectorSubcoreMesh(core_axis_name="c", subcore_axis_name="s")

    @functools.partial(
        pl.kernel,
        mesh=mesh,
        out_type=jax.ShapeDtypeStruct((n_tokens * _N_EXP,), jnp.float32),
        scratch_types=[
            pltpu.VMEM((chunk,), jnp.float32),
            pltpu.VMEM((chunk,), jnp.float32),
        ],
        compiler_params=pltpu.CompilerParams(needs_layout_passes=False),
    )
    def _router(rw_hbm, gates_hbm, buf_in, buf_out):
        wid = lax.axis_index("s") * _NC + lax.axis_index("c")
        base = wid * chunk
        pltpu.sync_copy(rw_hbm.at[pl.ds(base, chunk)], buf_in)

        lane = lax.iota(jnp.int32, _L)
        mask8 = lane < _K
        zero = jnp.zeros((_L,), jnp.float32)
        neg_inf = jnp.float32(-jnp.inf)

        def _sortd(v):
            return plsc.sort_key_val(v, v, descending=True)[0]

        def _merge(p, q):
            return _sortd(jnp.where(mask8, p, lax.rev(q, (0,))))

        @plsc.parallel_loop(0, rows_per_w, unroll=4)
        def row_body(r):
            off = r * _N_EXP
            v = [buf_in[pl.ds(off + j * _L, _L)] for j in range(4)]
            s0, s1, s2, s3 = (_sortd(vj) for vj in v)
            m3 = _merge(_merge(s0, s1), _merge(s2, s3))

            v0b = plsc.cummax(m3)
            t = jnp.where(mask8, -m3, neg_inf)
            v8b = -_bcast_max(t)

            e = jnp.where(mask8, jnp.exp(m3 - v0b), zero)
            s_e = plsc.cumsum(e)
            zb = plsc.cummax(lax.rev(s_e, (0,)))

            gt = [vj > v8b for vj in v]
            pc = [plsc.all_reduce_population_count(g) for g in gt]
            n_gt = pc[0] + pc[1] + pc[2] + pc[3]
            c = jnp.full((_L,), _K, jnp.int32) - n_gt

            eq = [vj == v8b for vj in v]
            pe = [plsc.all_reduce_population_count(q) for q in eq]
            prev = jnp.zeros((_L,), jnp.int32)
            for j in range(4):
                cs = plsc.cumsum(eq[j].astype(jnp.int32))
                keep = gt[j] | (eq[j] & ((prev + cs) <= c))
                out_j = jnp.where(keep, jnp.exp(v[j] - v0b) / zb, zero)
                buf_out[pl.ds(off + j * _L, _L)] = out_j
                prev = prev + pe[j]

        pltpu.sync_copy(buf_out, gates_hbm.at[pl.ds(base, chunk)])

    return _router(rw_flat)


def kernel(x, W, b):
    n_tokens = x.shape[0]
    rw = _tc_routing_weights(x, W, b, 0, n_tokens)
    gates_flat = _sc_gates(rw.reshape(-1), n_tokens)
    return (gates_flat.reshape(n_tokens, _N_EXP), rw)


# speedup vs baseline: 1.2511x; 1.0421x over previous
import functools

import jax
import jax.numpy as jnp
from jax import lax
from jax.experimental import pallas as pl
from jax.experimental.pallas import tpu as pltpu
from jax.experimental.pallas import tpu_sc as plsc

_D_MODEL = 4096
_N_EXP = 64
_K = 8
_BLOCK = 1024
_SPLIT = 2
_SUB = _BLOCK // _SPLIT

_NC = 2
_NS = 16
_NW = _NC * _NS
_L = 16


def _matmul_kernel(*refs):
    x_refs = refs[:_SPLIT]
    w_ref, b_ref, rw_ref = refs[_SPLIT:]
    w = w_ref[...]
    bias = b_ref[...]
    for s in range(_SPLIT):
        rw = lax.dot_general(
            x_refs[s][...], w, (((1,), (1,)), ((), ())),
            preferred_element_type=jnp.float32,
        ) + bias
        rw_ref[pl.ds(s * _SUB, _SUB), :] = rw


def _tc_routing_weights(x, W, b, row0, n_rows):
    grid = (n_rows // _BLOCK,)
    b2 = b.reshape(1, _N_EXP)
    sub0 = row0 // _SUB

    def _x_spec(s):
        return pl.BlockSpec(
            (_SUB, _D_MODEL), lambda i, s=s: (sub0 + _SPLIT * i + s, 0)
        )

    return pl.pallas_call(
        _matmul_kernel,
        grid=grid,
        in_specs=[_x_spec(s) for s in range(_SPLIT)] + [
            pl.BlockSpec((_N_EXP, _D_MODEL), lambda i: (0, 0)),
            pl.BlockSpec((1, _N_EXP), lambda i: (0, 0)),
        ],
        out_specs=pl.BlockSpec((_BLOCK, _N_EXP), lambda i: (i, 0)),
        out_shape=jax.ShapeDtypeStruct((n_rows, _N_EXP), jnp.float32),
        compiler_params=pltpu.CompilerParams(
            dimension_semantics=("parallel",),
        ),
    )(*([x] * _SPLIT + [W, b2]))


def _bcast_max(v):
    return plsc.cummax(lax.rev(plsc.cummax(v), (0,)))


def _sc_gates(rw_flat, n_tokens):
    rows_per_w = n_tokens // _NW
    chunk = rows_per_w * _N_EXP
    mesh = plsc.VectorSubcoreMesh(core_axis_name="c", subcore_axis_name="s")

    @functools.partial(
        pl.kernel,
        mesh=mesh,
        out_type=jax.ShapeDtypeStruct((n_tokens * _N_EXP,), jnp.float32),
        scratch_types=[
            pltpu.VMEM((chunk,), jnp.float32),
            pltpu.VMEM((chunk,), jnp.float32),
        ],
        compiler_params=pltpu.CompilerParams(needs_layout_passes=False),
    )
    def _router(rw_hbm, gates_hbm, buf_in, buf_out):
        wid = lax.axis_index("s") * _NC + lax.axis_index("c")
        base = wid * chunk
        pltpu.sync_copy(rw_hbm.at[pl.ds(base, chunk)], buf_in)

        lane = lax.iota(jnp.int32, _L)
        mask8 = lane < _K
        zero = jnp.zeros((_L,), jnp.float32)
        neg_inf = jnp.float32(-jnp.inf)

        def _sortd(v):
            return plsc.sort_key_val(v, v, descending=True)[0]

        def _merge(p, q):
            return _sortd(jnp.where(mask8, p, lax.rev(q, (0,))))

        @plsc.parallel_loop(0, rows_per_w, unroll=2)
        def row_body(r):
            off = r * _N_EXP
            v = [buf_in[pl.ds(off + j * _L, _L)] for j in range(4)]
            s0, s1, s2, s3 = (_sortd(vj) for vj in v)
            m3 = _merge(_merge(s0, s1), _merge(s2, s3))

            v0b = plsc.cummax(m3)
            t = jnp.where(mask8, -m3, neg_inf)
            v8b = -_bcast_max(t)

            e = jnp.where(mask8, jnp.exp(m3 - v0b), zero)
            s_e = plsc.cumsum(e)
            zb = plsc.cummax(lax.rev(s_e, (0,)))

            gt = [vj > v8b for vj in v]
            pc = [plsc.all_reduce_population_count(g) for g in gt]
            n_gt = pc[0] + pc[1] + pc[2] + pc[3]
            c = jnp.full((_L,), _K, jnp.int32) - n_gt

            eq = [vj == v8b for vj in v]
            pe = [plsc.all_reduce_population_count(q) for q in eq]
            prev = jnp.zeros((_L,), jnp.int32)
            for j in range(4):
                cs = plsc.cumsum(eq[j].astype(jnp.int32))
                keep = gt[j] | (eq[j] & ((prev + cs) <= c))
                out_j = jnp.where(keep, jnp.exp(v[j] - v0b) / zb, zero)
                buf_out[pl.ds(off + j * _L, _L)] = out_j
                prev = prev + pe[j]

        pltpu.sync_copy(buf_out, gates_hbm.at[pl.ds(base, chunk)])

    return _router(rw_flat)


def kernel(x, W, b):
    n_tokens = x.shape[0]
    rw = _tc_routing_weights(x, W, b, 0, n_tokens)
    gates_flat = _sc_gates(rw.reshape(-1), n_tokens)
    return (gates_flat.reshape(n_tokens, _N_EXP), rw)


# speedup vs baseline: 1.6242x; 1.2982x over previous
import jax
import jax.numpy as jnp
from jax import lax
from jax.experimental import pallas as pl
from jax.experimental.pallas import tpu as pltpu

_D_MODEL = 4096
_N_EXP = 64
_K = 8
_BLOCK = 1024
_SPLIT = 1
_SUB = _BLOCK // _SPLIT


def _topk_gates(rw):
    iota = lax.broadcasted_iota(jnp.int32, rw.shape, 1).astype(jnp.float32)
    work = rw
    v0 = None
    big = jnp.float32(2.0 * _N_EXP)
    neg_inf = jnp.float32(-jnp.inf)
    for t in range(_K):
        mx = jnp.max(work, axis=1, keepdims=True)
        if t == 0:
            v0 = mx
        masked = jnp.where(work == mx, iota, big)
        idx = jnp.min(masked, axis=1, keepdims=True)
        work = jnp.where(masked == idx, neg_inf, work)

    e_full = jnp.where(work == neg_inf, jnp.exp(rw - v0), 0.0)
    denom = jnp.sum(e_full, axis=1, keepdims=True)
    return e_full * (1.0 / denom)


def _router_kernel(*refs):
    x_refs = refs[:_SPLIT]
    w_ref, b_ref, gates_ref, rw_ref = refs[_SPLIT:]
    w = w_ref[...]
    bias = b_ref[...]
    for s in range(_SPLIT):
        rw = lax.dot_general(
            x_refs[s][...], w, (((1,), (1,)), ((), ())),
            preferred_element_type=jnp.float32,
        ) + bias
        rw_ref[pl.ds(s * _SUB, _SUB), :] = rw
        gates_ref[pl.ds(s * _SUB, _SUB), :] = _topk_gates(rw)


def kernel(x, W, b):
    n_tokens = x.shape[0]
    grid = (n_tokens // _BLOCK,)
    b2 = b.reshape(1, _N_EXP)

    def _x_spec(s):
        return pl.BlockSpec(
            (_SUB, _D_MODEL), lambda i, s=s: (_SPLIT * i + s, 0)
        )

    gates, rw = pl.pallas_call(
        _router_kernel,
        grid=grid,
        in_specs=[_x_spec(s) for s in range(_SPLIT)] + [
            pl.BlockSpec((_N_EXP, _D_MODEL), lambda i: (0, 0)),
            pl.BlockSpec((1, _N_EXP), lambda i: (0, 0)),
        ],
        out_specs=[
            pl.BlockSpec((_BLOCK, _N_EXP), lambda i: (i, 0)),
            pl.BlockSpec((_BLOCK, _N_EXP), lambda i: (i, 0)),
        ],
        out_shape=[
            jax.ShapeDtypeStruct((n_tokens, _N_EXP), jnp.float32),
            jax.ShapeDtypeStruct((n_tokens, _N_EXP), jnp.float32),
        ],
        compiler_params=pltpu.CompilerParams(
            dimension_semantics=("parallel",),
        ),
    )(*([x] * _SPLIT + [W, b2]))
    return (gates, rw)


# speedup vs baseline: 1.6790x; 1.0337x over previous
import jax
import jax.numpy as jnp
from jax import lax
from jax.experimental import pallas as pl
from jax.experimental.pallas import tpu as pltpu

_D_MODEL = 4096
_N_EXP = 64
_K = 8
_BLOCK = 1024
_SPLIT = 4
_SUB = _BLOCK // _SPLIT


def _topk_gates(rw):
    iota = lax.broadcasted_iota(jnp.int32, rw.shape, 1).astype(jnp.float32)
    work = rw
    v0 = None
    big = jnp.float32(2.0 * _N_EXP)
    neg_inf = jnp.float32(-jnp.inf)
    for t in range(_K):
        mx = jnp.max(work, axis=1, keepdims=True)
        if t == 0:
            v0 = mx
        masked = jnp.where(work == mx, iota, big)
        idx = jnp.min(masked, axis=1, keepdims=True)
        work = jnp.where(masked == idx, neg_inf, work)

    e_full = jnp.where(work == neg_inf, jnp.exp(rw - v0), 0.0)
    denom = jnp.sum(e_full, axis=1, keepdims=True)
    return e_full * (1.0 / denom)


def _router_kernel(*refs):
    x_refs = refs[:_SPLIT]
    w_ref, b_ref, gates_ref, rw_ref = refs[_SPLIT:]
    w = w_ref[...]
    bias = b_ref[...]
    for s in range(_SPLIT):
        rw = lax.dot_general(
            x_refs[s][...], w, (((1,), (1,)), ((), ())),
            preferred_element_type=jnp.float32,
        ) + bias
        rw_ref[pl.ds(s * _SUB, _SUB), :] = rw
        gates_ref[pl.ds(s * _SUB, _SUB), :] = _topk_gates(rw)


def kernel(x, W, b):
    n_tokens = x.shape[0]
    grid = (n_tokens // _BLOCK,)
    b2 = b.reshape(1, _N_EXP)

    def _x_spec(s):
        return pl.BlockSpec(
            (_SUB, _D_MODEL), lambda i, s=s: (_SPLIT * i + s, 0)
        )

    gates, rw = pl.pallas_call(
        _router_kernel,
        grid=grid,
        in_specs=[_x_spec(s) for s in range(_SPLIT)] + [
            pl.BlockSpec((_N_EXP, _D_MODEL), lambda i: (0, 0)),
            pl.BlockSpec((1, _N_EXP), lambda i: (0, 0)),
        ],
        out_specs=[
            pl.BlockSpec((_BLOCK, _N_EXP), lambda i: (i, 0)),
            pl.BlockSpec((_BLOCK, _N_EXP), lambda i: (i, 0)),
        ],
        out_shape=[
            jax.ShapeDtypeStruct((n_tokens, _N_EXP), jnp.float32),
            jax.ShapeDtypeStruct((n_tokens, _N_EXP), jnp.float32),
        ],
        compiler_params=pltpu.CompilerParams(
            dimension_semantics=("parallel",),
        ),
    )(*([x] * _SPLIT + [W, b2]))
    return (gates, rw)
